# baseline (speedup 1.0000x reference)
import functools

import jax
import jax.numpy as jnp
from jax import lax
from jax.experimental import pallas as pl
from jax.experimental.pallas import tpu as pltpu

E = 16
K = 2
D = 1024
H = 2048

R = 256
HT = 1024
NH = H // HT


def _gate_kernel(x_ref, gw_ref, idx_ref, sc_ref, imp_ref, load_ref, loss_ref):
    T = x_ref.shape[0]
    logits = jnp.dot(x_ref[...], gw_ref[...], preferred_element_type=jnp.float32)
    lane = lax.broadcasted_iota(jnp.int32, (T, E), 1)
    neg = jnp.float32(-3.0e38)

    m1 = jnp.max(logits, axis=1, keepdims=True)
    i1 = jnp.min(jnp.where(logits == m1, lane, E), axis=1, keepdims=True)
    hot1 = lane == i1
    l2 = jnp.where(hot1, neg, logits)
    m2 = jnp.max(l2, axis=1, keepdims=True)
    i2 = jnp.min(jnp.where(l2 == m2, lane, E), axis=1, keepdims=True)
    hot2 = lane == i2

    z = jnp.exp(m2 - m1)
    s1 = 1.0 / (1.0 + z)
    s2 = z / (1.0 + z)

    two = lax.broadcasted_iota(jnp.int32, (T, K), 1)
    idx_ref[...] = jnp.where(two == 0, i1, i2)
    sc_ref[...] = jnp.where(two == 0, s1, s2)

    sf = jnp.where(hot1, s1, 0.0) + jnp.where(hot2, s2, 0.0)
    imp = jnp.sum(sf, axis=0, keepdims=True)
    ld = jnp.sum(hot1.astype(jnp.float32) + hot2.astype(jnp.float32),
                 axis=0, keepdims=True)
    imp_ref[...] = imp
    load_ref[...] = ld

    def cv2(v):
        mu = jnp.mean(v, keepdims=True)
        var = jnp.mean((v - mu) ** 2, keepdims=True)
        return var / (mu * mu + 1e-10)

    loss_ref[...] = 0.01 * (cv2(imp) + cv2(ld))


def _run_gate(tok, gate_w):
    T = tok.shape[0]
    return pl.pallas_call(
        _gate_kernel,
        out_shape=[
            jax.ShapeDtypeStruct((T, K), jnp.int32),
            jax.ShapeDtypeStruct((T, K), jnp.float32),
            jax.ShapeDtypeStruct((1, E), jnp.float32),
            jax.ShapeDtypeStruct((1, E), jnp.float32),
            jax.ShapeDtypeStruct((1, 1), jnp.float32),
        ],
    )(tok, gate_w)


def _gmm_kernel(tile_ref, eid_ref, lo_ref, hi_ref,
                xs_ref, wg_ref, wu_ref, wd_ref, out_ref, acc_ref):
    g = pl.program_id(0)
    h = pl.program_id(1)
    xb = xs_ref[...]
    a = jnp.dot(xb, wg_ref[0], preferred_element_type=jnp.float32)
    b = jnp.dot(xb, wu_ref[0], preferred_element_type=jnp.float32)
    hh = (a * jax.nn.sigmoid(a) * b).astype(xb.dtype)
    part = jnp.dot(hh, wd_ref[0], preferred_element_type=jnp.float32)

    @pl.when(h == 0)
    def _():
        acc_ref[...] = part

    @pl.when(h != 0)
    def _():
        acc_ref[...] = acc_ref[...] + part

    @pl.when(h == NH - 1)
    def _():
        rows = tile_ref[g] * R + lax.broadcasted_iota(jnp.int32, (R, 1), 0)
        m = (rows >= lo_ref[g]) & (rows < hi_ref[g])
        out_ref[...] = jnp.where(m, acc_ref[...], out_ref[...])


def _run_gmm(x_sorted, w_gate, w_up, w_down, tile_id, eid, lo, hi, grid_g):
    TK = x_sorted.shape[0]
    grid_spec = pltpu.PrefetchScalarGridSpec(
        num_scalar_prefetch=4,
        grid=(grid_g, NH),
        in_specs=[
            pl.BlockSpec((R, D), lambda g, h, t, e, lo, hi: (t[g], 0)),
            pl.BlockSpec((1, D, HT), lambda g, h, t, e, lo, hi: (e[g], 0, h)),
            pl.BlockSpec((1, D, HT), lambda g, h, t, e, lo, hi: (e[g], 0, h)),
            pl.BlockSpec((1, HT, D), lambda g, h, t, e, lo, hi: (e[g], h, 0)),
        ],
        out_specs=pl.BlockSpec((R, D), lambda g, h, t, e, lo, hi: (t[g], 0)),
        scratch_shapes=[pltpu.VMEM((R, D), jnp.float32)],
    )
    return pl.pallas_call(
        _gmm_kernel,
        grid_spec=grid_spec,
        out_shape=jax.ShapeDtypeStruct((TK, D), jnp.float32),
    )(tile_id, eid, lo, hi, x_sorted, w_gate, w_up, w_down)


def kernel(x, gate_w, w_gate_proj, w_up_proj, w_down_proj):
    orig_shape = x.shape
    tok = x.reshape(-1, D)
    T = tok.shape[0]
    TK = T * K
    NT = TK // R
    G = NT + E - 1

    idx, sc, imp2, load2, loss2 = _run_gate(tok, gate_w)

    e_flat = idx.reshape(-1)
    onehot = (e_flat[:, None] == jnp.arange(E, dtype=jnp.int32)[None, :])
    oh32 = onehot.astype(jnp.int32)
    csum = jnp.cumsum(oh32, axis=0)
    pos = csum - oh32
    counts = csum[-1]
    offs = jnp.concatenate([jnp.zeros((1,), jnp.int32),
                            jnp.cumsum(counts)]).astype(jnp.int32)
    offr = jnp.sum(oh32 * offs[:E][None, :], axis=1)
    posr = jnp.sum(oh32 * pos, axis=1)
    dest = offr + posr

    boundaries = jnp.arange(NT, dtype=jnp.int32) * R
    first_e = jnp.sum((offs[1:][None, :] <= boundaries[:, None]).astype(jnp.int32), axis=1)
    last_e = jnp.sum((offs[1:][None, :] <= (boundaries + R - 1)[:, None]).astype(jnp.int32), axis=1)
    steps = last_e - first_e + 1
    tile_off = jnp.cumsum(steps) - steps
    g_ids = jnp.arange(G, dtype=jnp.int32)
    tile_id = (jnp.sum((tile_off[None, :] <= g_ids[:, None]).astype(jnp.int32), axis=1) - 1)
    tile_id = tile_id.astype(jnp.int32)
    eid = first_e[tile_id] + (g_ids - tile_off[tile_id])
    eid = jnp.minimum(eid, last_e[tile_id]).astype(jnp.int32)
    lo = jnp.maximum(offs[eid], boundaries[tile_id])
    hi = jnp.minimum(offs[eid + 1], boundaries[tile_id] + R)
    total = tile_off[-1] + steps[-1]
    hi = jnp.where(g_ids < total, hi, lo)

    perm = jnp.zeros((TK,), jnp.int32).at[dest].set(
        jnp.arange(TK, dtype=jnp.int32))
    x_sorted = tok.astype(jnp.bfloat16)[perm // K]

    out_sorted = _run_gmm(x_sorted,
                          w_gate_proj.astype(jnp.bfloat16),
                          w_up_proj.astype(jnp.bfloat16),
                          w_down_proj.astype(jnp.bfloat16),
                          tile_id, eid, lo, hi, G)

    out_pair = out_sorted[dest.reshape(T, K)]
    y = jnp.sum(out_pair * sc[:, :, None], axis=1)

    hidden = y.reshape(orig_shape)
    balance_loss = loss2[0, 0]
    num_dropped = jnp.array(0, dtype=jnp.int32)
    return hidden, balance_loss, num_dropped, load2[0], imp2[0]


# speedup vs baseline: 1.3684x; 1.3684x over previous
import functools

import jax
import jax.numpy as jnp
from jax import lax
from jax.experimental import pallas as pl
from jax.experimental.pallas import tpu as pltpu

E = 16
K = 2
D = 1024
H = 2048

R = 256
HT = 1024
NH = H // HT


def _gate_kernel(x_ref, gw_ref, idx_ref, sc_ref, imp_ref, load_ref, loss_ref):
    T = x_ref.shape[0]
    logits = jnp.dot(x_ref[...], gw_ref[...], preferred_element_type=jnp.float32)
    lane = lax.broadcasted_iota(jnp.int32, (T, E), 1)
    neg = jnp.float32(-3.0e38)

    m1 = jnp.max(logits, axis=1, keepdims=True)
    i1 = jnp.min(jnp.where(logits == m1, lane, E), axis=1, keepdims=True)
    hot1 = lane == i1
    l2 = jnp.where(hot1, neg, logits)
    m2 = jnp.max(l2, axis=1, keepdims=True)
    i2 = jnp.min(jnp.where(l2 == m2, lane, E), axis=1, keepdims=True)
    hot2 = lane == i2

    z = jnp.exp(m2 - m1)
    s1 = 1.0 / (1.0 + z)
    s2 = z / (1.0 + z)

    two = lax.broadcasted_iota(jnp.int32, (T, K), 1)
    idx_ref[...] = jnp.where(two == 0, i1, i2)
    sc_ref[...] = jnp.where(two == 0, s1, s2)

    sf = jnp.where(hot1, s1, 0.0) + jnp.where(hot2, s2, 0.0)
    imp = jnp.sum(sf, axis=0, keepdims=True)
    ld = jnp.sum(hot1.astype(jnp.float32) + hot2.astype(jnp.float32),
                 axis=0, keepdims=True)
    imp_ref[...] = imp
    load_ref[...] = ld

    def cv2(v):
        mu = jnp.mean(v, keepdims=True)
        var = jnp.mean((v - mu) ** 2, keepdims=True)
        return var / (mu * mu + 1e-10)

    loss_ref[...] = 0.01 * (cv2(imp) + cv2(ld))


def _run_gate(tok, gate_w):
    T = tok.shape[0]
    return pl.pallas_call(
        _gate_kernel,
        out_shape=[
            jax.ShapeDtypeStruct((T, K), jnp.int32),
            jax.ShapeDtypeStruct((T, K), jnp.float32),
            jax.ShapeDtypeStruct((1, E), jnp.float32),
            jax.ShapeDtypeStruct((1, E), jnp.float32),
            jax.ShapeDtypeStruct((1, 1), jnp.float32),
        ],
    )(tok, gate_w)


def _gmm_kernel(tile_ref, eid_ref, lo_ref, hi_ref,
                xs_ref, wg_ref, wu_ref, wd_ref, out_ref, acc_ref):
    h = pl.program_id(0)
    g = pl.program_id(1)
    xb = xs_ref[...]
    a = jnp.dot(xb, wg_ref[0], preferred_element_type=jnp.float32)
    b = jnp.dot(xb, wu_ref[0], preferred_element_type=jnp.float32)
    hh = a * jax.nn.sigmoid(a) * b
    part = jnp.dot(hh, wd_ref[0], preferred_element_type=jnp.float32)

    rows = tile_ref[g] * R + lax.broadcasted_iota(jnp.int32, (R, 1), 0)
    m = (rows >= lo_ref[g]) & (rows < hi_ref[g])
    start = tile_ref[g] * R

    @pl.when(h == 0)
    def _():
        cur = acc_ref[pl.ds(start, R), :]
        acc_ref[pl.ds(start, R), :] = jnp.where(m, part, cur)

    @pl.when(h != 0)
    def _():
        cur = acc_ref[pl.ds(start, R), :]
        acc_ref[pl.ds(start, R), :] = cur + jnp.where(m, part, 0.0)

    @pl.when(h == NH - 1)
    def _():
        out_ref[...] = jnp.where(m, acc_ref[pl.ds(start, R), :], out_ref[...])


def _run_gmm(x_sorted, w_gate, w_up, w_down, tile_id, eid, lo, hi, grid_g):
    TK = x_sorted.shape[0]
    grid_spec = pltpu.PrefetchScalarGridSpec(
        num_scalar_prefetch=4,
        grid=(NH, grid_g),
        in_specs=[
            pl.BlockSpec((R, D), lambda h, g, t, e, lo, hi: (t[g], 0)),
            pl.BlockSpec((1, D, HT), lambda h, g, t, e, lo, hi: (e[g], 0, h)),
            pl.BlockSpec((1, D, HT), lambda h, g, t, e, lo, hi: (e[g], 0, h)),
            pl.BlockSpec((1, HT, D), lambda h, g, t, e, lo, hi: (e[g], h, 0)),
        ],
        out_specs=pl.BlockSpec(
            (R, D),
            lambda h, g, t, e, lo, hi: (jnp.where(h == NH - 1, t[g], 0), 0)),
        scratch_shapes=[pltpu.VMEM((TK, D), jnp.float32)],
    )
    return pl.pallas_call(
        _gmm_kernel,
        grid_spec=grid_spec,
        out_shape=jax.ShapeDtypeStruct((TK, D), jnp.float32),
    )(tile_id, eid, lo, hi, x_sorted, w_gate, w_up, w_down)


def kernel(x, gate_w, w_gate_proj, w_up_proj, w_down_proj):
    orig_shape = x.shape
    tok = x.reshape(-1, D)
    T = tok.shape[0]
    TK = T * K
    NT = TK // R
    G = NT + E - 1

    idx, sc, imp2, load2, loss2 = _run_gate(tok, gate_w)

    e_flat = idx.reshape(-1)
    onehot = (e_flat[:, None] == jnp.arange(E, dtype=jnp.int32)[None, :])
    oh32 = onehot.astype(jnp.int32)
    csum = jnp.cumsum(oh32, axis=0)
    pos = csum - oh32
    counts = csum[-1]
    offs = jnp.concatenate([jnp.zeros((1,), jnp.int32),
                            jnp.cumsum(counts)]).astype(jnp.int32)
    offr = jnp.sum(oh32 * offs[:E][None, :], axis=1)
    posr = jnp.sum(oh32 * pos, axis=1)
    dest = offr + posr

    boundaries = jnp.arange(NT, dtype=jnp.int32) * R
    first_e = jnp.sum((offs[1:][None, :] <= boundaries[:, None]).astype(jnp.int32), axis=1)
    last_e = jnp.sum((offs[1:][None, :] <= (boundaries + R - 1)[:, None]).astype(jnp.int32), axis=1)
    steps = last_e - first_e + 1
    tile_off = jnp.cumsum(steps) - steps
    g_ids = jnp.arange(G, dtype=jnp.int32)
    tile_id = (jnp.sum((tile_off[None, :] <= g_ids[:, None]).astype(jnp.int32), axis=1) - 1)
    tile_id = tile_id.astype(jnp.int32)
    eid = first_e[tile_id] + (g_ids - tile_off[tile_id])
    eid = jnp.minimum(eid, last_e[tile_id]).astype(jnp.int32)
    lo = jnp.maximum(offs[eid], boundaries[tile_id])
    hi = jnp.minimum(offs[eid + 1], boundaries[tile_id] + R)
    total = tile_off[-1] + steps[-1]
    hi = jnp.where(g_ids < total, hi, lo)

    perm = jnp.zeros((TK,), jnp.int32).at[dest].set(
        jnp.arange(TK, dtype=jnp.int32))
    x_sorted = tok[perm // K]

    out_sorted = _run_gmm(x_sorted, w_gate_proj, w_up_proj, w_down_proj,
                          tile_id, eid, lo, hi, G)

    out_pair = out_sorted[dest.reshape(T, K)]
    y = jnp.sum(out_pair * sc[:, :, None], axis=1)

    hidden = y.reshape(orig_shape)
    balance_loss = loss2[0, 0]
    num_dropped = jnp.array(0, dtype=jnp.int32)
    return hidden, balance_loss, num_dropped, load2[0], imp2[0]


# speedup vs baseline: 1.4005x; 1.0235x over previous
import functools

import jax
import jax.numpy as jnp
from jax import lax
from jax.experimental import pallas as pl
from jax.experimental.pallas import tpu as pltpu

E = 16
K = 2
D = 1024
H = 2048

R = 128
HT = 1024
NH = H // HT


def _gate_kernel(x_ref, gw_ref, idx_ref, sc_ref, imp_ref, load_ref, loss_ref):
    T = x_ref.shape[0]
    logits = jnp.dot(x_ref[...], gw_ref[...], preferred_element_type=jnp.float32)
    lane = lax.broadcasted_iota(jnp.int32, (T, E), 1)
    neg = jnp.float32(-3.0e38)

    m1 = jnp.max(logits, axis=1, keepdims=True)
    i1 = jnp.min(jnp.where(logits == m1, lane, E), axis=1, keepdims=True)
    hot1 = lane == i1
    l2 = jnp.where(hot1, neg, logits)
    m2 = jnp.max(l2, axis=1, keepdims=True)
    i2 = jnp.min(jnp.where(l2 == m2, lane, E), axis=1, keepdims=True)
    hot2 = lane == i2

    z = jnp.exp(m2 - m1)
    s1 = 1.0 / (1.0 + z)
    s2 = z / (1.0 + z)

    two = lax.broadcasted_iota(jnp.int32, (T, K), 1)
    idx_ref[...] = jnp.where(two == 0, i1, i2)
    sc_ref[...] = jnp.where(two == 0, s1, s2)

    sf = jnp.where(hot1, s1, 0.0) + jnp.where(hot2, s2, 0.0)
    imp = jnp.sum(sf, axis=0, keepdims=True)
    ld = jnp.sum(hot1.astype(jnp.float32) + hot2.astype(jnp.float32),
                 axis=0, keepdims=True)
    imp_ref[...] = imp
    load_ref[...] = ld

    def cv2(v):
        mu = jnp.mean(v, keepdims=True)
        var = jnp.mean((v - mu) ** 2, keepdims=True)
        return var / (mu * mu + 1e-10)

    loss_ref[...] = 0.01 * (cv2(imp) + cv2(ld))


def _run_gate(tok, gate_w):
    T = tok.shape[0]
    return pl.pallas_call(
        _gate_kernel,
        out_shape=[
            jax.ShapeDtypeStruct((T, K), jnp.int32),
            jax.ShapeDtypeStruct((T, K), jnp.float32),
            jax.ShapeDtypeStruct((1, E), jnp.float32),
            jax.ShapeDtypeStruct((1, E), jnp.float32),
            jax.ShapeDtypeStruct((1, 1), jnp.float32),
        ],
    )(tok, gate_w)


def _gmm_kernel(eid_ref, xs_ref, wg_ref, wu_ref, wd_ref, out_ref):
    xb = xs_ref[...]
    a = jnp.dot(xb, wg_ref[0], preferred_element_type=jnp.float32)
    b = jnp.dot(xb, wu_ref[0], preferred_element_type=jnp.float32)
    hh = a * jax.nn.sigmoid(a) * b
    out_ref[...] = jnp.dot(hh, wd_ref[0], preferred_element_type=jnp.float32)


def _run_gmm(x_sorted, w_gate, w_up, w_down, eid, grid_g):
    PAD = x_sorted.shape[0]
    grid_spec = pltpu.PrefetchScalarGridSpec(
        num_scalar_prefetch=1,
        grid=(grid_g,),
        in_specs=[
            pl.BlockSpec((R, D), lambda g, e: (g, 0)),
            pl.BlockSpec((1, D, H), lambda g, e: (e[g], 0, 0)),
            pl.BlockSpec((1, D, H), lambda g, e: (e[g], 0, 0)),
            pl.BlockSpec((1, H, D), lambda g, e: (e[g], 0, 0)),
        ],
        out_specs=pl.BlockSpec((R, D), lambda g, e: (g, 0)),
    )
    return pl.pallas_call(
        _gmm_kernel,
        grid_spec=grid_spec,
        out_shape=jax.ShapeDtypeStruct((PAD, D), jnp.float32),
    )(eid, x_sorted, w_gate, w_up, w_down)


def kernel(x, gate_w, w_gate_proj, w_up_proj, w_down_proj):
    orig_shape = x.shape
    tok = x.reshape(-1, D)
    T = tok.shape[0]
    TK = T * K
    NT = TK // R
    G = NT + E
    PAD = G * R

    idx, sc, imp2, load2, loss2 = _run_gate(tok, gate_w)

    e_flat = idx.reshape(-1)
    onehot = (e_flat[:, None] == jnp.arange(E, dtype=jnp.int32)[None, :])
    oh32 = onehot.astype(jnp.int32)
    csum = jnp.cumsum(oh32, axis=0)
    pos = csum - oh32
    counts = csum[-1]
    tiles_per_e = (counts + R - 1) // R
    cum_tiles = jnp.cumsum(tiles_per_e)
    pad_offs = (cum_tiles - tiles_per_e) * R
    offr = jnp.sum(oh32 * pad_offs[None, :], axis=1)
    posr = jnp.sum(oh32 * pos, axis=1)
    dest = offr + posr

    g_ids = jnp.arange(G, dtype=jnp.int32)
    eid = jnp.sum((cum_tiles[None, :] <= g_ids[:, None]).astype(jnp.int32),
                  axis=1)
    eid = jnp.minimum(eid, E - 1).astype(jnp.int32)

    perm = jnp.zeros((PAD,), jnp.int32).at[dest].set(
        jnp.arange(TK, dtype=jnp.int32))
    x_sorted = tok[perm // K]

    out_sorted = _run_gmm(x_sorted, w_gate_proj, w_up_proj, w_down_proj,
                          eid, G)

    out_pair = out_sorted[dest.reshape(T, K)]
    y = jnp.sum(out_pair * sc[:, :, None], axis=1)

    hidden = y.reshape(orig_shape)
    balance_loss = loss2[0, 0]
    num_dropped = jnp.array(0, dtype=jnp.int32)
    return hidden, balance_loss, num_dropped, load2[0], imp2[0]
